# baseline (speedup 1.0000x reference)
import functools

import jax
import jax.numpy as jnp
from jax import lax
from jax.experimental import pallas as pl
from jax.experimental.pallas import tpu as pltpu
from jax.experimental.pallas import tpu_sc as plsc

B = 16384
F = 32
L = 16
NC = 2
NS = 16
NW = NC * NS
BPW = B // NW
CHUNK = 256
NCHUNK = BPW // CHUNK


def _sc_body(time_hbm, item_hbm, tf_hbm, if_hbm, out_hbm,
             tidx_v, iidx_v, tb_v, ib_v, trows_v, irows_v, out_v,
             sem_t, sem_i):
    wid = lax.axis_index("s") * NC + lax.axis_index("c")
    base = wid * BPW

    pltpu.sync_copy(time_hbm.at[pl.ds(base, BPW)], tidx_v)
    pltpu.sync_copy(item_hbm.at[pl.ds(base, BPW)], iidx_v)

    def blockids(j, carry):
        s = pl.ds(j * L, L)
        tb_v[s] = tidx_v[s] >> 2
        ib_v[s] = iidx_v[s] >> 2
        return carry

    lax.fori_loop(0, BPW // L, blockids, 0)

    lane = lax.iota(jnp.int32, L)

    for c in range(NCHUNK):
        ct = pltpu.async_copy(
            tf_hbm.at[tb_v.at[pl.ds(c * CHUNK, CHUNK)]], trows_v, sem_t)
        ci = pltpu.async_copy(
            if_hbm.at[ib_v.at[pl.ds(c * CHUNK, CHUNK)]], irows_v, sem_i)
        ct.wait()
        ci.wait()

        def group(g, carry):
            rows = g * L + lane
            s = pl.ds(c * CHUNK + g * L, L)
            qt = (tidx_v[s] & 3) << 5
            qi = (iidx_v[s] & 3) << 5
            acc = jnp.zeros((L,), jnp.float32)
            for f in range(F):
                tv = plsc.load_gather(trows_v, [rows, qt + f])
                iv = plsc.load_gather(irows_v, [rows, qi + f])
                acc = acc + tv * iv
            out_v[s] = acc
            return carry

        lax.fori_loop(0, CHUNK // L, group, 0)

    pltpu.sync_copy(out_v, out_hbm.at[pl.ds(base, BPW)])


@jax.jit
def _run(time, item, time_factor4, item_factor4):
    kern = pl.kernel(
        _sc_body,
        out_type=jax.ShapeDtypeStruct((B,), jnp.float32),
        mesh=plsc.VectorSubcoreMesh(core_axis_name="c", subcore_axis_name="s"),
        compiler_params=pltpu.CompilerParams(needs_layout_passes=False),
        scratch_types=[
            pltpu.VMEM((BPW,), jnp.int32),
            pltpu.VMEM((BPW,), jnp.int32),
            pltpu.VMEM((BPW,), jnp.int32),
            pltpu.VMEM((BPW,), jnp.int32),
            pltpu.VMEM((CHUNK, 4 * F), jnp.float32),
            pltpu.VMEM((CHUNK, 4 * F), jnp.float32),
            pltpu.VMEM((BPW,), jnp.float32),
            pltpu.SemaphoreType.DMA,
            pltpu.SemaphoreType.DMA,
        ],
    )
    return kern(time, item, time_factor4, item_factor4)


def kernel(time, item, time_factor, item_factor, lag_factor):
    del lag_factor
    tf4 = time_factor.reshape(-1, 4 * F)
    if4 = item_factor.reshape(-1, 4 * F)
    return _run(time.astype(jnp.int32), item.astype(jnp.int32), tf4, if4)


# speedup vs baseline: 2.2756x; 2.2756x over previous
import functools

import jax
import jax.numpy as jnp
from jax import lax
from jax.experimental import pallas as pl
from jax.experimental.pallas import tpu as pltpu
from jax.experimental.pallas import tpu_sc as plsc

B = 16384
F = 32
L = 16
TW = 128
NC = 2
NS = 16
NW = NC * NS
BPW = B // NW
K = 16
NCHUNK = BPW // K


def _gather_phase(tbl_hbm, idx_v, blk_v, emb_v, sem):
    lane = lax.iota(jnp.int32, L)

    def chunk(c, carry):
        idx_vec = idx_v[pl.ds(c * K, L)]
        cols = (idx_vec >> 7) << 7
        for r in range(K):
            col = pl.multiple_of(cols[r], TW)
            pltpu.async_copy(
                tbl_hbm.at[pl.ds(0, F), pl.ds(col, TW)], blk_v.at[r], sem)

        def drain(r, carry2):
            pltpu.make_async_copy(
                tbl_hbm.at[pl.ds(0, F), pl.ds(0, TW)], blk_v.at[r], sem
            ).wait()
            return carry2

        lax.fori_loop(0, K, drain, 0, unroll=2)

        rows = lane
        lanes = idx_vec & (TW - 1)
        for f in range(F):
            fv = jnp.full((L,), f, jnp.int32)
            emb_v[f, pl.ds(c * K, L)] = plsc.load_gather(
                blk_v, [rows, fv, lanes])
        return carry

    lax.fori_loop(0, NCHUNK, chunk, 0)


def _sc_body(time_hbm, item_hbm, tf_hbm, if_hbm, out_hbm,
             tidx_v, iidx_v, blk_v, temb_v, iemb_v, out_v,
             sem):
    wid = lax.axis_index("s") * NC + lax.axis_index("c")
    base = wid * BPW

    pltpu.sync_copy(time_hbm.at[pl.ds(base, BPW)], tidx_v)
    pltpu.sync_copy(item_hbm.at[pl.ds(base, BPW)], iidx_v)

    _gather_phase(tf_hbm, tidx_v, blk_v, temb_v, sem)
    _gather_phase(if_hbm, iidx_v, blk_v, iemb_v, sem)

    def dot(g, carry):
        s = pl.ds(g * L, L)
        acc = jnp.zeros((L,), jnp.float32)
        for f in range(F):
            acc = acc + temb_v[f, s] * iemb_v[f, s]
        out_v[s] = acc
        return carry

    lax.fori_loop(0, BPW // L, dot, 0)

    pltpu.sync_copy(out_v, out_hbm.at[pl.ds(base, BPW)])


@jax.jit
def _run(time, item, tfT, ifT):
    kern = pl.kernel(
        _sc_body,
        out_type=jax.ShapeDtypeStruct((B,), jnp.float32),
        mesh=plsc.VectorSubcoreMesh(core_axis_name="c", subcore_axis_name="s"),
        compiler_params=pltpu.CompilerParams(needs_layout_passes=False),
        scratch_types=[
            pltpu.VMEM((BPW,), jnp.int32),
            pltpu.VMEM((BPW,), jnp.int32),
            pltpu.VMEM((K, F, TW), jnp.float32),
            pltpu.VMEM((F, BPW), jnp.float32),
            pltpu.VMEM((F, BPW), jnp.float32),
            pltpu.VMEM((BPW,), jnp.float32),
            pltpu.SemaphoreType.DMA,
        ],
    )
    return kern(time, item, tfT, ifT)


def kernel(time, item, time_factor, item_factor, lag_factor):
    del lag_factor
    return _run(time.astype(jnp.int32), item.astype(jnp.int32),
                time_factor.T, item_factor.T)
